# Initial kernel scaffold; baseline (speedup 1.0000x reference)
import jax
import jax.numpy as jnp
from jax.experimental import pallas as pl


def kernel(z, idx):
    raise NotImplementedError("write your pallas kernel here")


# speedup vs baseline: 2.4871x; 2.4871x over previous
import functools

import jax
import jax.numpy as jnp
from jax import lax
from jax.experimental import pallas as pl
from jax.experimental.pallas import tpu as pltpu
from jax.experimental.pallas import tpu_sc as plsc

_NUM_CORES = 2
_NUM_SUBCORES = 16
_NUM_WORKERS = _NUM_CORES * _NUM_SUBCORES

_CHUNK = 2048


def _gather_flat(table, idx_flat):
    n = idx_flat.shape[0]
    d = table.shape[1]
    per_w = n // _NUM_WORKERS
    n_chunks = per_w // _CHUNK
    assert per_w * _NUM_WORKERS == n and n_chunks * _CHUNK == per_w

    mesh = plsc.VectorSubcoreMesh(core_axis_name="c", subcore_axis_name="s")

    @functools.partial(
        pl.kernel,
        mesh=mesh,
        out_type=jax.ShapeDtypeStruct((n, d), jnp.float32),
        scratch_types=[
            pltpu.VMEM((_CHUNK,), jnp.int32),
            pltpu.VMEM((_CHUNK, d), jnp.float32),
            pltpu.SemaphoreType.DMA,
        ],
        compiler_params=pltpu.CompilerParams(use_tc_tiling_on_sc=False),
    )
    def k(table_hbm, idx_hbm, out_hbm, idx_v, rows_v, sem):
        wid = lax.axis_index("s") * _NUM_CORES + lax.axis_index("c")
        base = wid * per_w

        def body(j, carry):
            off = base + j * _CHUNK
            pltpu.sync_copy(idx_hbm.at[pl.ds(off, _CHUNK)], idx_v)
            pltpu.async_copy(table_hbm.at[idx_v], rows_v, sem).wait()
            pltpu.sync_copy(rows_v, out_hbm.at[pl.ds(off, _CHUNK)])
            return carry

        lax.fori_loop(0, n_chunks, body, 0)

    return k(table, idx_flat)


def kernel(z, idx):
    b, h = idx.shape
    idx_flat = idx.reshape(b * h).astype(jnp.int32)
    out = _gather_flat(z, idx_flat)
    return out.reshape(b, h, z.shape[1])


# speedup vs baseline: 2.5328x; 1.0184x over previous
import functools

import jax
import jax.numpy as jnp
from jax import lax
from jax.experimental import pallas as pl
from jax.experimental.pallas import tpu as pltpu
from jax.experimental.pallas import tpu_sc as plsc

_NUM_CORES = 2
_NUM_SUBCORES = 16
_NUM_WORKERS = _NUM_CORES * _NUM_SUBCORES

_CHUNK = 2048
_NBUF = 2


def _gather_flat(table, idx_flat):
    n = idx_flat.shape[0]
    d = table.shape[1]
    per_w = n // _NUM_WORKERS
    n_chunks = per_w // _CHUNK
    n_groups = n_chunks // _NBUF
    assert per_w * _NUM_WORKERS == n and n_groups * _NBUF * _CHUNK == per_w

    mesh = plsc.VectorSubcoreMesh(core_axis_name="c", subcore_axis_name="s")

    @functools.partial(
        pl.kernel,
        mesh=mesh,
        out_type=jax.ShapeDtypeStruct((n, d), jnp.float32),
        scratch_types=(
            [pltpu.VMEM((_CHUNK,), jnp.int32) for _ in range(_NBUF)]
            + [pltpu.VMEM((_CHUNK, d), jnp.float32) for _ in range(_NBUF)]
            + [pltpu.SemaphoreType.DMA for _ in range(3 * _NBUF)]
        ),
        compiler_params=pltpu.CompilerParams(use_tc_tiling_on_sc=False),
    )
    def k(table_hbm, idx_hbm, out_hbm, *scratch):
        idx_v = scratch[:_NBUF]
        rows_v = scratch[_NBUF:2 * _NBUF]
        sem_i = scratch[2 * _NBUF:3 * _NBUF]
        sem_g = scratch[3 * _NBUF:4 * _NBUF]
        sem_s = scratch[4 * _NBUF:5 * _NBUF]

        wid = lax.axis_index("s") * _NUM_CORES + lax.axis_index("c")
        base = wid * per_w

        def idx_copy(b, j):
            return pltpu.make_async_copy(
                idx_hbm.at[pl.ds(base + j * _CHUNK, _CHUNK)], idx_v[b], sem_i[b])

        def gather_copy(b):
            return pltpu.make_async_copy(table_hbm.at[idx_v[b]], rows_v[b], sem_g[b])

        def store_copy(b, j):
            return pltpu.make_async_copy(
                rows_v[b], out_hbm.at[pl.ds(base + j * _CHUNK, _CHUNK)], sem_s[b])

        for b in range(_NBUF):
            idx_copy(b, b).start()

        def group(g, carry):
            for b in range(_NBUF):
                j = g * _NBUF + b

                @pl.when(g > 0)
                def _():
                    store_copy(b, j - _NBUF).wait()

                idx_copy(b, j).wait()
                gather_copy(b).start()
                gather_copy(b).wait()
                store_copy(b, j).start()

                @pl.when(j + _NBUF < n_chunks)
                def _():
                    idx_copy(b, j + _NBUF).start()
            return carry

        lax.fori_loop(0, n_groups, group, 0)

        for b in range(_NBUF):
            store_copy(b, n_chunks - _NBUF + b).wait()

    return k(table, idx_flat)


def kernel(z, idx):
    b, h = idx.shape
    idx_flat = idx.reshape(b * h).astype(jnp.int32)
    out = _gather_flat(z, idx_flat)
    return out.reshape(b, h, z.shape[1])


# speedup vs baseline: 3.8671x; 1.5268x over previous
import functools

import jax
import jax.numpy as jnp
from jax import lax
from jax.experimental import pallas as pl
from jax.experimental.pallas import tpu as pltpu
from jax.experimental.pallas import tpu_sc as plsc

_NUM_CORES = 2
_NUM_SUBCORES = 16
_NUM_WORKERS = 32
_CHUNK = 1024
_LANES = 16


def _gather(table, idx_t):
    h_len, p_len = idx_t.shape
    d = table.shape[1]
    n_chunks = p_len // _CHUNK
    n_slabs_max = (h_len + _NUM_WORKERS - 1) // _NUM_WORKERS
    slab_stride = (d // 8) * p_len * 8
    db_stride = p_len * 8
    chunk_out = _CHUNK * d
    half_chunk = chunk_out // 2

    mesh = plsc.VectorSubcoreMesh(core_axis_name="c", subcore_axis_name="s")

    @functools.partial(
        pl.kernel,
        mesh=mesh,
        out_type=jax.ShapeDtypeStruct((p_len * h_len * d,), jnp.float32),
        scratch_types=(
            [pltpu.VMEM((p_len,), jnp.int32)]
            + [pltpu.VMEM((_CHUNK, d), jnp.float32) for _ in range(2)]
            + [pltpu.VMEM((chunk_out,), jnp.float32) for _ in range(2)]
            + [pltpu.SemaphoreType.DMA for _ in range(4)]
        ),
        compiler_params=pltpu.CompilerParams(use_tc_tiling_on_sc=False, needs_layout_passes=False),
    )
    def k(table_hbm, idx_hbm, out_hbm, idx_s, r0, r1, t0, t1, gs0, gs1,
          ss0, ss1):
        rows = (r0, r1)
        trans = (t0, t1)
        sem_g = (gs0, gs1)
        sem_s = (ss0, ss1)

        wid = lax.axis_index("s") * _NUM_CORES + lax.axis_index("c")

        lane = lax.iota(jnp.int32, _LANES)
        const_off = (lane // 8) * half_chunk + (lane % 8) * 128

        def transpose_chunk(b):
            def body(o, carry):
                row = rows[b][o]
                scal = (o // 128) * 1024 + lax.rem(o, 128)
                plsc.store_scatter(trans[b], [const_off + scal], row)
                return carry
            lax.fori_loop(0, _CHUNK, body, 0, unroll=8)

        def gather_copy(b, pj):
            return pltpu.make_async_copy(
                table_hbm.at[idx_s.at[pl.ds(pj * _CHUNK, _CHUNK)]],
                rows[b], sem_g[b])

        def store_copy(b, h, pj, db):
            dst = out_hbm.at[pl.ds(
                h * slab_stride + db * db_stride + pj * half_chunk,
                half_chunk)]
            return pltpu.make_async_copy(
                trans[b].at[pl.ds(db * half_chunk, half_chunk)], dst,
                sem_s[b])

        for t in range(n_slabs_max):
            h = wid + t * _NUM_WORKERS

            @pl.when(h < h_len)
            def _():
                pltpu.sync_copy(idx_hbm.at[h], idx_s)
                gather_copy(0, 0).start()

                def chunk_group(g, carry):
                    for b in range(2):
                        pj = g * 2 + b

                        @pl.when(pj + 1 < n_chunks)
                        def _():
                            gather_copy(1 - b, pj + 1).start()

                        gather_copy(b, pj).wait()

                        @pl.when(t * n_chunks + pj >= 2)
                        def _():
                            for db in range(2):
                                store_copy(b, 0, 0, db).wait()

                        transpose_chunk(b)
                        for db in range(2):
                            store_copy(b, h, pj, db).start()
                    return carry

                lax.fori_loop(0, n_chunks // 2, chunk_group, 0)

        for b in range(2):
            for db in range(2):
                store_copy(b, 0, 0, db).wait()

    return k(table, idx_t)


def kernel(z, idx):
    p, h = idx.shape
    d = z.shape[1]
    idx_t = jnp.transpose(idx.astype(jnp.int32))
    out_flat = _gather(z, idx_t)
    out5d = out_flat.reshape(h, d // 8, p // 128, 8, 128)
    return out5d.transpose(2, 4, 0, 1, 3).reshape(p, h, d)


# speedup vs baseline: 4.6093x; 1.1919x over previous
import functools

import jax
import jax.numpy as jnp
from jax import lax
from jax.experimental import pallas as pl
from jax.experimental.pallas import tpu as pltpu
from jax.experimental.pallas import tpu_sc as plsc

_NUM_CORES = 2
_NUM_SUBCORES = 16
_NUM_WORKERS = 32
_CHUNK = 1024
_LANES = 16


def _gather(table, idx_t):
    h_len, p_len = idx_t.shape
    d = table.shape[1]
    n_chunks = p_len // _CHUNK
    n_slabs_max = (h_len + _NUM_WORKERS - 1) // _NUM_WORKERS
    slab_stride = (d // 8) * p_len * 8
    db_stride = p_len * 8
    chunk_out = _CHUNK * d
    half_chunk = chunk_out // 2

    mesh = plsc.VectorSubcoreMesh(core_axis_name="c", subcore_axis_name="s")

    @functools.partial(
        pl.kernel,
        mesh=mesh,
        out_type=jax.ShapeDtypeStruct((p_len * h_len * d,), jnp.float32),
        scratch_types=(
            [pltpu.VMEM((p_len,), jnp.int32)]
            + [pltpu.VMEM((_CHUNK, d), jnp.float32) for _ in range(2)]
            + [pltpu.VMEM((chunk_out,), jnp.float32) for _ in range(2)]
            + [pltpu.SemaphoreType.DMA for _ in range(4)]
        ),
        compiler_params=pltpu.CompilerParams(use_tc_tiling_on_sc=False, needs_layout_passes=False),
    )
    def k(table_hbm, idx_hbm, out_hbm, idx_s, r0, r1, t0, t1, gs0, gs1,
          ss0, ss1):
        rows = (r0, r1)
        trans = (t0, t1)
        sem_g = (gs0, gs1)
        sem_s = (ss0, ss1)

        wid = lax.axis_index("s") * _NUM_CORES + lax.axis_index("c")

        lane = lax.iota(jnp.int32, _LANES)
        const_off = (lane // 8) * half_chunk + (lane % 8) * 128

        def transpose_chunk(b):
            def body(pg, carry):
                pvec = pg * 16 + lane
                scal = (pg // 8) * 1024 + lax.rem(pg, 8) * 16
                for dd in range(d):
                    col = plsc.load_gather(rows[b], [pvec, lane * 0 + dd])
                    off = (dd // 8) * half_chunk + (dd % 8) * 128
                    trans[b][pl.ds(off + scal, 16)] = col
                return carry
            lax.fori_loop(0, _CHUNK // 16, body, 0, unroll=2)

        def gather_copy(b, pj):
            return pltpu.make_async_copy(
                table_hbm.at[idx_s.at[pl.ds(pj * _CHUNK, _CHUNK)]],
                rows[b], sem_g[b])

        def store_copy(b, h, pj, db):
            dst = out_hbm.at[pl.ds(
                h * slab_stride + db * db_stride + pj * half_chunk,
                half_chunk)]
            return pltpu.make_async_copy(
                trans[b].at[pl.ds(db * half_chunk, half_chunk)], dst,
                sem_s[b])

        for t in range(n_slabs_max):
            h = wid + t * _NUM_WORKERS

            @pl.when(h < h_len)
            def _():
                pltpu.sync_copy(idx_hbm.at[h], idx_s)
                gather_copy(0, 0).start()

                def chunk_group(g, carry):
                    for b in range(2):
                        pj = g * 2 + b

                        @pl.when(pj + 1 < n_chunks)
                        def _():
                            gather_copy(1 - b, pj + 1).start()

                        gather_copy(b, pj).wait()

                        @pl.when(t * n_chunks + pj >= 2)
                        def _():
                            for db in range(2):
                                store_copy(b, 0, 0, db).wait()

                        transpose_chunk(b)
                        for db in range(2):
                            store_copy(b, h, pj, db).start()
                    return carry

                lax.fori_loop(0, n_chunks // 2, chunk_group, 0)

        for b in range(2):
            for db in range(2):
                store_copy(b, 0, 0, db).wait()

    return k(table, idx_t)


def kernel(z, idx):
    p, h = idx.shape
    d = z.shape[1]
    idx_t = jnp.transpose(idx.astype(jnp.int32))
    out_flat = _gather(z, idx_t)
    out5d = out_flat.reshape(h, d // 8, p // 128, 8, 128)
    return out5d.transpose(2, 4, 0, 1, 3).reshape(p, h, d)
